# baseline (speedup 1.0000x reference)
import functools

import jax
import jax.numpy as jnp
from jax import lax
from jax.experimental import pallas as pl
from jax.experimental.pallas import tpu as pltpu
from jax.experimental.pallas import tpu_sc as plsc

VOCAB = 100000
D = 1024
BATCH = 4
SEQ = 8192

NC = 2
NS = 16
NW = NC * NS

POS_PER_W = SEQ // NW
CHUNK = 32
NCHUNK = POS_PER_W // CHUNK
LANES = 16
SLICES = D // LANES


def _make_sc_kernel():
    mesh = plsc.VectorSubcoreMesh(core_axis_name="c", subcore_axis_name="s")

    @functools.partial(
        pl.kernel,
        mesh=mesh,
        out_type=jax.ShapeDtypeStruct((BATCH * SEQ, D), jnp.float32),
        scratch_types=[
            pltpu.VMEM((BATCH * NCHUNK, CHUNK), jnp.int32),
            pltpu.VMEM((CHUNK, D), jnp.float32),
            pltpu.VMEM((CHUNK, D), jnp.float32),
            pltpu.SemaphoreType.DMA,
        ],
    )
    def emb_kernel(x2d_hbm, table_hbm, pe_hbm, out_hbm, idx_v, pe_v, rows_v, sem):
        wid = lax.axis_index("s") * NC + lax.axis_index("c")
        pos0 = wid * POS_PER_W

        for b in range(BATCH):
            pltpu.sync_copy(
                x2d_hbm.at[pl.ds(b * (SEQ // CHUNK) + wid * NCHUNK, NCHUNK)],
                idx_v.at[pl.ds(b * NCHUNK, NCHUNK)],
            )

        def chunk_body(c, carry):
            pltpu.sync_copy(pe_hbm.at[pl.ds(pos0 + c * CHUNK, CHUNK)], pe_v)

            def batch_body(b, carry2):
                pltpu.async_copy(
                    table_hbm.at[idx_v.at[b * NCHUNK + c]], rows_v, sem
                ).wait()

                def add_row(r, carry3):
                    for j in range(SLICES):
                        sl = pl.ds(j * LANES, LANES)
                        rows_v[r, sl] += pe_v[r, sl]
                    return carry3

                lax.fori_loop(0, CHUNK, add_row, 0, unroll=False)

                row0 = b * SEQ + pos0 + c * CHUNK
                pltpu.sync_copy(rows_v, out_hbm.at[pl.ds(row0, CHUNK)])
                return carry2

            lax.fori_loop(0, BATCH, batch_body, 0, unroll=False)
            return carry

        lax.fori_loop(0, NCHUNK, chunk_body, 0, unroll=False)

    return emb_kernel


_emb_kernel = _make_sc_kernel()


def kernel(x, table, pe):
    x2d = x.reshape(BATCH * SEQ // CHUNK, CHUNK)
    out = _emb_kernel(x2d, table, pe)
    return out.reshape(BATCH, SEQ, D)


# speedup vs baseline: 1.1308x; 1.1308x over previous
import functools

import jax
import jax.numpy as jnp
from jax import lax
from jax.experimental import pallas as pl
from jax.experimental.pallas import tpu as pltpu
from jax.experimental.pallas import tpu_sc as plsc

VOCAB = 100000
D = 1024
BATCH = 4
SEQ = 8192

NC = 2
NS = 16
NW = NC * NS

POS_PER_W = SEQ // NW
CHUNK = 16
NCHUNK = POS_PER_W // CHUNK
LANES = 16
SLICES = D // LANES


def _make_sc_kernel():
    mesh = plsc.VectorSubcoreMesh(core_axis_name="c", subcore_axis_name="s")

    @functools.partial(
        pl.kernel,
        mesh=mesh,
        out_type=jax.ShapeDtypeStruct((BATCH * SEQ, D), jnp.float32),
        scratch_types=[
            pltpu.VMEM((BATCH * NCHUNK, CHUNK), jnp.int32),
            pltpu.VMEM((CHUNK, D), jnp.float32),
            pltpu.VMEM((CHUNK, D), jnp.float32),
            pltpu.VMEM((CHUNK, D), jnp.float32),
            pltpu.VMEM((CHUNK, D), jnp.float32),
            pltpu.VMEM((CHUNK, D), jnp.float32),
            pltpu.VMEM((CHUNK, D), jnp.float32),
            pltpu.SemaphoreType.DMA,
            pltpu.SemaphoreType.DMA,
            pltpu.SemaphoreType.DMA,
            pltpu.SemaphoreType.DMA,
            pltpu.SemaphoreType.DMA,
            pltpu.SemaphoreType.DMA,
            pltpu.SemaphoreType.DMA,
            pltpu.SemaphoreType.DMA,
            pltpu.SemaphoreType.DMA,
            pltpu.SemaphoreType.DMA,
        ],
    )
    def emb_kernel(x2d_hbm, table_hbm, pe_hbm, out_hbm, idx_v,
                   rows0, rows1, rows2, rows3, pe0, pe1,
                   gs0, gs1, gs2, gs3, ss0, ss1, ss2, ss3, ps0, ps1):
        rows = (rows0, rows1, rows2, rows3)
        gsem = (gs0, gs1, gs2, gs3)
        ssem = (ss0, ss1, ss2, ss3)
        pebuf = (pe0, pe1)
        psem = (ps0, ps1)

        wid = lax.axis_index("s") * NC + lax.axis_index("c")
        pos0 = wid * POS_PER_W

        for b in range(BATCH):
            pltpu.sync_copy(
                x2d_hbm.at[pl.ds(b * (SEQ // CHUNK) + wid * NCHUNK, NCHUNK)],
                idx_v.at[pl.ds(b * NCHUNK, NCHUNK)],
            )

        def g_copy(c, b):
            return pltpu.make_async_copy(
                table_hbm.at[idx_v.at[b * NCHUNK + c]], rows[b], gsem[b])

        def s_copy(c, b):
            return pltpu.make_async_copy(
                rows[b], out_hbm.at[pl.ds(b * SEQ + pos0 + c * CHUNK, CHUNK)],
                ssem[b])

        def pe_copy(c, h):
            return pltpu.make_async_copy(
                pe_hbm.at[pl.ds(pos0 + c * CHUNK, CHUNK)], pebuf[h], psem[h])

        def add_item(rows_ref, pe_ref):
            def row_body(r, carry):
                for j in range(SLICES):
                    sl = pl.ds(j * LANES, LANES)
                    plsc.addupdate(rows_ref.at[r, sl], pe_ref[r, sl])
                return carry
            lax.fori_loop(0, CHUNK, row_body, 0, unroll=False)

        pe_copy(0, 0).start()
        g_copy(0, 0).start()

        def group_pair(gg, carry):
            for half in range(2):
                g = 2 * gg + half
                pe_copy(g, half).wait()

                @pl.when(g < NCHUNK - 1)
                def _():
                    pe_copy(g + 1, 1 - half).start()

                for b in range(BATCH):
                    g_copy(g, b).wait()
                    add_item(rows[b], pebuf[half])
                    s_copy(g, b).start()
                    if b < BATCH - 1:
                        @pl.when(g >= 1)
                        def _():
                            s_copy(g - 1, b + 1).wait()

                        g_copy(g, b + 1).start()
                    else:
                        s_copy(g, 0).wait()

                        @pl.when(g < NCHUNK - 1)
                        def _():
                            g_copy(g + 1, 0).start()
            return carry

        lax.fori_loop(0, NCHUNK // 2, group_pair, 0, unroll=False)

        for b in range(1, BATCH):
            s_copy(NCHUNK - 1, b).wait()

    return emb_kernel


_emb_kernel = _make_sc_kernel()


def kernel(x, table, pe):
    x2d = x.reshape(BATCH * SEQ // CHUNK, CHUNK)
    out = _emb_kernel(x2d, table, pe)
    return out.reshape(BATCH, SEQ, D)


# speedup vs baseline: 1.1466x; 1.0139x over previous
import functools

import jax
import jax.numpy as jnp
from jax import lax
from jax.experimental import pallas as pl
from jax.experimental.pallas import tpu as pltpu
from jax.experimental.pallas import tpu_sc as plsc

VOCAB = 100000
D = 1024
BATCH = 4
SEQ = 8192

NC = 2
NS = 16
NW = NC * NS

POS_PER_W = SEQ // NW
CHUNK = 16
NCHUNK = POS_PER_W // CHUNK
NITEM = BATCH * NCHUNK
LANES = 16
SLICES = D // LANES


def _make_sc_kernel():
    mesh = plsc.VectorSubcoreMesh(core_axis_name="c", subcore_axis_name="s")

    @functools.partial(
        pl.kernel,
        mesh=mesh,
        out_type=jax.ShapeDtypeStruct((BATCH * SEQ, D), jnp.float32),
        scratch_types=[
            pltpu.VMEM((BATCH * NCHUNK, CHUNK), jnp.int32),
            pltpu.VMEM((BATCH, CHUNK, D), jnp.float32),
            pltpu.VMEM((2, CHUNK, D), jnp.float32),
            pltpu.SemaphoreType.DMA((BATCH,)),
            pltpu.SemaphoreType.DMA((BATCH,)),
            pltpu.SemaphoreType.DMA((2,)),
        ],
    )
    def emb_kernel(x2d_hbm, table_hbm, pe_hbm, out_hbm,
                   idx_v, rows_v, pe_v, gsem, ssem, psem):
        wid = lax.axis_index("s") * NC + lax.axis_index("c")
        pos0 = wid * POS_PER_W

        for b in range(BATCH):
            pltpu.sync_copy(
                x2d_hbm.at[pl.ds(b * (SEQ // CHUNK) + wid * NCHUNK, NCHUNK)],
                idx_v.at[pl.ds(b * NCHUNK, NCHUNK)],
            )

        def g_copy(c, b):
            return pltpu.make_async_copy(
                table_hbm.at[idx_v.at[b * NCHUNK + c]], rows_v.at[b],
                gsem.at[b])

        def s_copy(c, b):
            return pltpu.make_async_copy(
                rows_v.at[b],
                out_hbm.at[pl.ds(b * SEQ + pos0 + c * CHUNK, CHUNK)],
                ssem.at[b])

        def pe_copy(c):
            return pltpu.make_async_copy(
                pe_hbm.at[pl.ds(pos0 + c * CHUNK, CHUNK)],
                pe_v.at[lax.rem(c, 2)], psem.at[lax.rem(c, 2)])

        pe_copy(0).start()
        g_copy(0, 0).start()

        def item(t, carry):
            c = t // BATCH
            b = lax.rem(t, BATCH)

            @pl.when(b == 0)
            def _():
                pe_copy(c).wait()

                @pl.when(c < NCHUNK - 1)
                def _():
                    pe_copy(c + 1).start()

            g_copy(c, b).wait()

            rows_ref = rows_v.at[b]
            pe_ref = pe_v.at[lax.rem(c, 2)]

            @plsc.parallel_loop(0, CHUNK, 1, unroll=2)
            def _(r):
                for j in range(SLICES):
                    sl = pl.ds(j * LANES, LANES)
                    plsc.addupdate(rows_ref.at[r, sl], pe_ref[r, sl])

            s_copy(c, b).start()

            tn = t + 1
            cn = tn // BATCH
            bn = lax.rem(tn, BATCH)

            @pl.when(t >= 3)
            def _():
                tp = t - 3
                s_copy(tp // BATCH, lax.rem(tp, BATCH)).wait()

            @pl.when(t < NITEM - 1)
            def _():
                g_copy(cn, bn).start()

            return carry

        lax.fori_loop(0, NITEM, item, 0, unroll=False)

        for dt in range(NITEM - 3, NITEM):
            s_copy(dt // BATCH, dt % BATCH).wait()

    return emb_kernel


_emb_kernel = _make_sc_kernel()


def kernel(x, table, pe):
    x2d = x.reshape(BATCH * SEQ // CHUNK, CHUNK)
    out = _emb_kernel(x2d, table, pe)
    return out.reshape(BATCH, SEQ, D)


# speedup vs baseline: 2.0129x; 1.7556x over previous
import functools

import jax
import jax.numpy as jnp
from jax import lax
from jax.experimental import pallas as pl
from jax.experimental.pallas import tpu as pltpu
from jax.experimental.pallas import tpu_sc as plsc

VOCAB = 100000
D = 1024
BATCH = 4
SEQ = 8192

NC = 2
NS = 16
NW = NC * NS

POS_PER_W = SEQ // NW
CHUNK = 16
NCHUNK = POS_PER_W // CHUNK
NITEM = BATCH * NCHUNK
LANES = 16
SLICES = D // LANES


def _make_sc_kernel():
    mesh = plsc.VectorSubcoreMesh(core_axis_name="c", subcore_axis_name="s")

    @functools.partial(
        pl.kernel,
        mesh=mesh,
        out_type=jax.ShapeDtypeStruct((BATCH * SEQ, D), jnp.float32),
        scratch_types=[
            pltpu.VMEM((BATCH * NCHUNK, CHUNK), jnp.int32),
            pltpu.VMEM((BATCH, CHUNK, D), jnp.float32),
            pltpu.VMEM((2, CHUNK, D), jnp.float32),
            pltpu.SemaphoreType.DMA((BATCH,)),
            pltpu.SemaphoreType.DMA((BATCH,)),
            pltpu.SemaphoreType.DMA((2,)),
        ],
    )
    def emb_kernel(x2d_hbm, table_hbm, pe_hbm, out_hbm,
                   idx_v, rows_v, pe_v, gsem, ssem, psem):
        wid = lax.axis_index("s") * NC + lax.axis_index("c")
        pos0 = wid * POS_PER_W

        for b in range(BATCH):
            pltpu.sync_copy(
                x2d_hbm.at[pl.ds(b * (SEQ // CHUNK) + wid * NCHUNK, NCHUNK)],
                idx_v.at[pl.ds(b * NCHUNK, NCHUNK)],
            )

        def g_copy(c, b):
            return pltpu.make_async_copy(
                table_hbm.at[idx_v.at[b * NCHUNK + c]], rows_v.at[b],
                gsem.at[b])

        def s_copy(c, b):
            return pltpu.make_async_copy(
                rows_v.at[b],
                out_hbm.at[pl.ds(b * SEQ + pos0 + c * CHUNK, CHUNK)],
                ssem.at[b])

        def pe_copy(c):
            return pltpu.make_async_copy(
                pe_hbm.at[pl.ds(pos0 + c * CHUNK, CHUNK)],
                pe_v.at[lax.rem(c, 2)], psem.at[lax.rem(c, 2)])

        pe_copy(0).start()
        g_copy(0, 0).start()
        g_copy(0, 1).start()

        def item(t, carry):
            c = t // BATCH
            b = lax.rem(t, BATCH)

            @pl.when(b == 0)
            def _():
                pe_copy(c).wait()

                @pl.when(c < NCHUNK - 1)
                def _():
                    pe_copy(c + 1).start()

            @pl.when(t >= 2)
            def _():
                tp = t - 2
                s_copy(tp // BATCH, lax.rem(tp, BATCH)).wait()

            @pl.when(t < NITEM - 2)
            def _():
                tn = t + 2
                g_copy(tn // BATCH, lax.rem(tn, BATCH)).start()

            g_copy(c, b).wait()

            rows_ref = rows_v.at[b]
            pe_ref = pe_v.at[lax.rem(c, 2)]

            @plsc.parallel_loop(0, CHUNK, 1, unroll=2)
            def _(r):
                for j in range(SLICES):
                    sl = pl.ds(j * LANES, LANES)
                    plsc.addupdate(rows_ref.at[r, sl], pe_ref[r, sl])

            s_copy(c, b).start()
            return carry

        lax.fori_loop(0, NITEM, item, 0, unroll=False)

        for dt in range(NITEM - 2, NITEM):
            s_copy(dt // BATCH, dt % BATCH).wait()

    return emb_kernel


_emb_kernel = _make_sc_kernel()


def kernel(x, table, pe):
    x2d = x.reshape(BATCH * SEQ // CHUNK, CHUNK)
    out = _emb_kernel(x2d, table, pe)
    return out.reshape(BATCH, SEQ, D)


# speedup vs baseline: 2.0150x; 1.0010x over previous
import functools

import jax
import jax.numpy as jnp
from jax import lax
from jax.experimental import pallas as pl
from jax.experimental.pallas import tpu as pltpu
from jax.experimental.pallas import tpu_sc as plsc

VOCAB = 100000
D = 1024
BATCH = 4
SEQ = 8192

NC = 2
NS = 16
NW = NC * NS

POS_PER_W = SEQ // NW
CHUNK = 16
NCHUNK = POS_PER_W // CHUNK
NITEM = BATCH * NCHUNK
RING = 5
LOOKAHEAD = 3
LANES = 16
SLICES = D // LANES


def _make_sc_kernel():
    mesh = plsc.VectorSubcoreMesh(core_axis_name="c", subcore_axis_name="s")

    @functools.partial(
        pl.kernel,
        mesh=mesh,
        out_type=jax.ShapeDtypeStruct((BATCH * SEQ, D), jnp.float32),
        scratch_types=[
            pltpu.VMEM((BATCH * NCHUNK, CHUNK), jnp.int32),
            pltpu.VMEM((RING, CHUNK, D), jnp.float32),
            pltpu.VMEM((2, CHUNK, D), jnp.float32),
            pltpu.SemaphoreType.DMA((RING,)),
            pltpu.SemaphoreType.DMA((RING,)),
            pltpu.SemaphoreType.DMA((2,)),
        ],
    )
    def emb_kernel(x2d_hbm, table_hbm, pe_hbm, out_hbm,
                   idx_v, rows_v, pe_v, gsem, ssem, psem):
        wid = lax.axis_index("s") * NC + lax.axis_index("c")
        pos0 = wid * POS_PER_W

        for b in range(BATCH):
            pltpu.sync_copy(
                x2d_hbm.at[pl.ds(b * (SEQ // CHUNK) + wid * NCHUNK, NCHUNK)],
                idx_v.at[pl.ds(b * NCHUNK, NCHUNK)],
            )

        def g_copy(t):
            c, b, slot = t // BATCH, lax.rem(t, BATCH), lax.rem(t, RING)
            return pltpu.make_async_copy(
                table_hbm.at[idx_v.at[b * NCHUNK + c]], rows_v.at[slot],
                gsem.at[slot])

        def s_copy(t):
            c, b, slot = t // BATCH, lax.rem(t, BATCH), lax.rem(t, RING)
            return pltpu.make_async_copy(
                rows_v.at[slot],
                out_hbm.at[pl.ds(b * SEQ + pos0 + c * CHUNK, CHUNK)],
                ssem.at[slot])

        def pe_copy(c):
            return pltpu.make_async_copy(
                pe_hbm.at[pl.ds(pos0 + c * CHUNK, CHUNK)],
                pe_v.at[lax.rem(c, 2)], psem.at[lax.rem(c, 2)])

        pe_copy(0).start()
        for tp in range(LOOKAHEAD):
            g_copy(tp).start()

        def item(t, carry):
            c = t // BATCH
            b = lax.rem(t, BATCH)

            @pl.when(b == 0)
            def _():
                pe_copy(c).wait()

                @pl.when(c < NCHUNK - 1)
                def _():
                    pe_copy(c + 1).start()

            @pl.when(t >= RING - LOOKAHEAD)
            def _():
                s_copy(t - (RING - LOOKAHEAD)).wait()

            @pl.when(t < NITEM - LOOKAHEAD)
            def _():
                g_copy(t + LOOKAHEAD).start()

            g_copy(t).wait()

            slot = lax.rem(t, RING)
            rows_ref = rows_v.at[slot]
            pe_ref = pe_v.at[lax.rem(c, 2)]

            @plsc.parallel_loop(0, CHUNK, 1, unroll=2)
            def _(r):
                for j in range(SLICES):
                    sl = pl.ds(j * LANES, LANES)
                    plsc.addupdate(rows_ref.at[r, sl], pe_ref[r, sl])

            s_copy(t).start()
            return carry

        lax.fori_loop(0, NITEM, item, 0, unroll=False)

        for dt in range(NITEM - (RING - LOOKAHEAD), NITEM):
            s_copy(dt).wait()

    return emb_kernel


_emb_kernel = _make_sc_kernel()


def kernel(x, table, pe):
    x2d = x.reshape(BATCH * SEQ // CHUNK, CHUNK)
    out = _emb_kernel(x2d, table, pe)
    return out.reshape(BATCH, SEQ, D)
